# baseline (speedup 1.0000x reference)
import functools

import jax
import jax.numpy as jnp
from jax import lax
from jax.experimental import pallas as pl
from jax.experimental.pallas import tpu as pltpu
from jax.experimental.pallas import tpu_sc as plsc

K_NN = 32


def _topk_body(xf_ref, w1_ref, idx_ref, wt_ref):
    PB = xf_ref.shape[0]
    N = xf_ref.shape[1]
    DC = xf_ref.shape[2]
    C = w1_ref.shape[0]
    D = DC // C

    w1 = w1_ref[...]
    w1n = w1 / jnp.maximum(
        jnp.sqrt(jnp.sum(w1 * w1, axis=1, keepdims=True)), 1e-12)
    wt_ref[...] = jnp.concatenate([w1n.T] * D, axis=1)

    big = jnp.int32(1 << 30)
    inf = jnp.float32(jnp.inf)

    xfs = [xf_ref[p] for p in range(PB)]
    adjs = []
    for p in range(PB):
        xf = xfs[p]
        sq = jnp.sum(xf * xf, axis=1, keepdims=True)
        inner = lax.dot_general(xf, xf, (((1,), (1,)), ((), ())),
                                preferred_element_type=jnp.float32)
        adjs.append(sq - 2.0 * inner + sq.T)

    def step(k, carry):
        adjs, iaccs = carry
        ek = (lax.broadcasted_iota(jnp.int32, (1, K_NN), 1) == k
              ).astype(jnp.float32)
        new_adjs, new_iaccs = [], []
        for p in range(PB):
            adj = adjs[p]
            iota = lax.broadcasted_iota(jnp.int32, (N, N), 1)
            rowmin = jnp.min(adj, axis=1, keepdims=True)
            tied = adj == rowmin
            idxm = jnp.min(jnp.where(tied, iota, big), axis=1, keepdims=True)
            onehot = iota == idxm
            new_iaccs.append(iaccs[p] + idxm.astype(jnp.float32) * ek)
            new_adjs.append(jnp.where(onehot, inf, adj))
        return tuple(new_adjs), tuple(new_iaccs)

    iacc0 = tuple(jnp.zeros((N, K_NN), dtype=jnp.float32) for _ in range(PB))
    _, iaccs = lax.fori_loop(0, K_NN, step, (tuple(adjs), iacc0))

    b = pl.program_id(0)
    for p in range(PB):
        gbase = (b * PB + p) * N
        idx_ref[p] = iaccs[p].astype(jnp.int32) + gbase


def _topk(xf, w1):
    B, N, DC = xf.shape
    C = w1.shape[0]
    PB = 4
    return pl.pallas_call(
        _topk_body,
        grid=(B // PB,),
        in_specs=[
            pl.BlockSpec((PB, N, DC), lambda b: (b, 0, 0)),
            pl.BlockSpec((C, K_NN), lambda b: (0, 0)),
        ],
        out_specs=[
            pl.BlockSpec((PB, N, K_NN), lambda b: (b, 0, 0)),
            pl.BlockSpec((K_NN, DC), lambda b: (0, 0)),
        ],
        out_shape=[
            jax.ShapeDtypeStruct((B, N, K_NN), jnp.int32),
            jax.ShapeDtypeStruct((K_NN, DC), jnp.float32),
        ],
    )(xf, w1)


def _sc_gather_combine(xf_flat, idx_flat, wt):
    P, DC = xf_flat.shape
    NW = 32
    CH = P // NW
    mesh = plsc.VectorSubcoreMesh(core_axis_name="c", subcore_axis_name="s")

    @functools.partial(
        pl.kernel, mesh=mesh,
        out_type=jax.ShapeDtypeStruct((P, DC), jnp.float32),
        scratch_types=[
            pltpu.VMEM((CH * K_NN,), jnp.int32),
            pltpu.VMEM((K_NN, DC), jnp.float32),
            pltpu.VMEM((K_NN, DC), jnp.float32),
            pltpu.VMEM((CH, DC), jnp.float32),
            pltpu.SemaphoreType.DMA,
        ],
    )
    def sck(xf_hbm, idx_hbm, wt_hbm, out_hbm, idx_v, rows_v, wt_v, out_v, sem):
        wid = lax.axis_index("s") * 2 + lax.axis_index("c")
        base = wid * CH
        pltpu.sync_copy(idx_hbm.at[pl.ds(base * K_NN, CH * K_NN)], idx_v)
        pltpu.sync_copy(wt_hbm, wt_v)

        def point(i, carry):
            pltpu.async_copy(
                xf_hbm.at[idx_v.at[pl.ds(i * K_NN, K_NN)]], rows_v, sem
            ).wait()
            for j in range(DC // 16):
                s = j * 16
                acc = rows_v[0, pl.ds(s, 16)] * wt_v[0, pl.ds(s, 16)]
                for k in range(1, K_NN):
                    acc = acc + rows_v[k, pl.ds(s, 16)] * wt_v[k, pl.ds(s, 16)]
                out_v[i, pl.ds(s, 16)] = acc
            return carry

        lax.fori_loop(0, CH, point, 0)
        pltpu.sync_copy(out_v, out_hbm.at[pl.ds(base, CH)])

    return sck(xf_flat, idx_flat, wt)


def _mix_body(acc_ref, w2_ref, out_ref):
    w2 = w2_ref[...]
    w2n = w2 / jnp.maximum(
        jnp.sqrt(jnp.sum(w2 * w2, axis=0, keepdims=True)), 1e-12)
    out_ref[...] = lax.dot_general(acc_ref[...], w2n,
                                   (((1,), (0,)), ((), ())),
                                   preferred_element_type=jnp.float32)


def _mix(acc2d, w2):
    M, C = acc2d.shape
    O = w2.shape[1]
    return pl.pallas_call(
        _mix_body,
        out_shape=jax.ShapeDtypeStruct((M, O), jnp.float32),
    )(acc2d, w2)


def kernel(x, w1, w2, conv_w, conv_b):
    B, N, D, C = x.shape
    O = w2.shape[1]
    xf = x.reshape(B, N, D * C)
    idx, wt = _topk(xf, w1)
    out1 = _sc_gather_combine(xf.reshape(B * N, D * C),
                              idx.reshape(B * N * K_NN), wt)
    out2 = _mix(out1.reshape(B * N * D, C), w2)
    return out2.reshape(B, N, D, O)


# speedup vs baseline: 1.8788x; 1.8788x over previous
import jax
import jax.numpy as jnp
from jax import lax
from jax.experimental import pallas as pl

K_NN = 32


def _body(xf_ref, w1_ref, w2_ref, out_ref):
    PB = xf_ref.shape[0]
    N = xf_ref.shape[1]
    DC = xf_ref.shape[2]
    C = w1_ref.shape[0]
    D = DC // C

    w1 = w1_ref[...]
    w1n = w1 / jnp.maximum(
        jnp.sqrt(jnp.sum(w1 * w1, axis=1, keepdims=True)), 1e-12)
    wt = jnp.concatenate([w1n.T] * D, axis=1)
    w2 = w2_ref[...]
    w2n = w2 / jnp.maximum(
        jnp.sqrt(jnp.sum(w2 * w2, axis=0, keepdims=True)), 1e-12)

    big = jnp.int32(1 << 30)
    inf = jnp.float32(jnp.inf)
    kiota = lax.broadcasted_iota(jnp.int32, (K_NN, DC), 0)

    xfs = [xf_ref[p] for p in range(PB)]
    adjs = []
    for p in range(PB):
        xf = xfs[p]
        sq = jnp.sum(xf * xf, axis=1, keepdims=True)
        inner = lax.dot_general(xf, xf, (((1,), (1,)), ((), ())),
                                preferred_element_type=jnp.float32)
        adjs.append(sq - 2.0 * inner + sq.T)

    def step(k, carry):
        adjs, accs = carry
        wk = jnp.sum(jnp.where(kiota == k, wt, 0.0), axis=0, keepdims=True)
        new_adjs, new_accs = [], []
        for p in range(PB):
            adj, acc = adjs[p], accs[p]
            iota = lax.broadcasted_iota(jnp.int32, (N, N), 1)
            idxm = jnp.argmin(adj, axis=1).astype(jnp.int32)[:, None]
            onehot = iota == idxm
            g = lax.dot_general(onehot.astype(jnp.float32), xfs[p],
                                (((1,), (0,)), ((), ())),
                                preferred_element_type=jnp.float32)
            new_accs.append(acc + g * wk)
            new_adjs.append(jnp.where(onehot, inf, adj))
        return tuple(new_adjs), tuple(new_accs)

    acc0 = tuple(jnp.zeros((N, DC), dtype=jnp.float32) for _ in range(PB))
    _, accs = lax.fori_loop(0, K_NN, step, (tuple(adjs), acc0))

    for p in range(PB):
        pieces = []
        for d in range(D):
            pieces.append(lax.dot_general(accs[p][:, d * C:(d + 1) * C], w2n,
                                          (((1,), (0,)), ((), ())),
                                          preferred_element_type=jnp.float32))
        out_ref[p] = jnp.concatenate(pieces, axis=1)


def kernel(x, w1, w2, conv_w, conv_b):
    B, N, D, C = x.shape
    O = w2.shape[1]
    PB = 4
    xf = x.reshape(B, N, D * C)
    out = pl.pallas_call(
        _body,
        grid=(B // PB,),
        in_specs=[
            pl.BlockSpec((PB, N, D * C), lambda b: (b, 0, 0)),
            pl.BlockSpec((C, K_NN), lambda b: (0, 0)),
            pl.BlockSpec((C, O), lambda b: (0, 0)),
        ],
        out_specs=pl.BlockSpec((PB, N, D * O), lambda b: (b, 0, 0)),
        out_shape=jax.ShapeDtypeStruct((B, N, D * O), jnp.float32),
    )(xf, w1, w2)
    return out.reshape(B, N, D, O)
